# baseline (speedup 1.0000x reference)
import functools

import jax
import jax.numpy as jnp
from jax import lax
from jax.experimental import pallas as pl
from jax.experimental.pallas import tpu as pltpu
from jax.experimental.pallas import tpu_sc as plsc

N_ROWS = 1_000_000
K = 64
B = 16384

_CHUNK = 128


def _build():
    info = plsc.get_sparse_core_info()
    nw = info.num_cores * info.num_subcores
    b_per_w = B // nw
    n_chunks = b_per_w // _CHUNK
    mesh = plsc.VectorSubcoreMesh(core_axis_name="c", subcore_axis_name="s")

    @functools.partial(
        pl.kernel,
        mesh=mesh,
        out_type=(
            jax.ShapeDtypeStruct((B, K), jnp.float32),
            jax.ShapeDtypeStruct((B, K), jnp.float32),
        ),
        scratch_types=[
            pltpu.VMEM((b_per_w,), jnp.int32),
            pltpu.VMEM((b_per_w, K), jnp.float32),
            pltpu.VMEM((b_per_w, K), jnp.float32),
            pltpu.SemaphoreType.DMA,
        ],
        compiler_params=pltpu.CompilerParams(use_tc_tiling_on_sc=False),
    )
    def k(idx_hbm, mu_hbm, ls_hbm, mu_out, ls_out, idx_v, mu_v, ls_v, sem):
        wid = lax.axis_index("s") * info.num_cores + lax.axis_index("c")
        base = wid * b_per_w
        pltpu.sync_copy(idx_hbm.at[pl.ds(base, b_per_w)], idx_v)
        copies = []
        for j in range(n_chunks):
            o = j * _CHUNK
            idx_c = idx_v.at[pl.ds(o, _CHUNK)]
            copies.append(
                pltpu.async_copy(
                    mu_hbm.at[idx_c], mu_v.at[pl.ds(o, _CHUNK)], sem
                )
            )
            copies.append(
                pltpu.async_copy(
                    ls_hbm.at[idx_c], ls_v.at[pl.ds(o, _CHUNK)], sem
                )
            )
        for c in copies:
            c.wait()
        pltpu.sync_copy(mu_v, mu_out.at[pl.ds(base, b_per_w)])
        pltpu.sync_copy(ls_v, ls_out.at[pl.ds(base, b_per_w)])

    return k


_gather = _build()


def kernel(indices, mu, log_sigma):
    return _gather(indices.astype(jnp.int32), mu, log_sigma)
